# baseline (speedup 1.0000x reference)
import functools

import jax
import jax.numpy as jnp
from jax import lax
from jax.experimental import pallas as pl
from jax.experimental.pallas import tpu as pltpu
from jax.experimental.pallas import tpu_sc as plsc

N = 10000
E = 320000
D = 128
T = 4

NC = 2
NS = 16
NW = NC * NS
BATCH = 128
NB = -(-E // (NW * BATCH))
EPW = NB * BATCH
EPAD = EPW * NW
ZCH = 128
RPT = -(-N // (NS * ZCH)) * ZCH
NPAD = RPT * NS
CHUNKS = RPT // ZCH

BN = 400


def _sc_aggregate(x, vecflat, gidx, didx, zrow, dpad):
    @functools.partial(
        pl.kernel,
        out_type=(
            jax.ShapeDtypeStruct((NC, NPAD, D), jnp.float32),
            jax.ShapeDtypeStruct((NC, 3, NPAD, D), jnp.float32),
        ),
        mesh=plsc.VectorSubcoreMesh(
            core_axis_name="c", subcore_axis_name="s",
            num_cores=NC, num_subcores=NS),
        scratch_types=[
            pltpu.VMEM_SHARED((NPAD, D), jnp.float32),
            pltpu.VMEM((NB, BATCH), jnp.int32),
            pltpu.VMEM((NB, BATCH), jnp.int32),
            pltpu.VMEM((BATCH, D), jnp.float32),
            pltpu.SemaphoreType.DMA,
        ],
    )
    def k(x_hbm, vf_hbm, gidx_hbm, didx_hbm, zrow_hbm, dpad_hbm,
          aggx_hbm, aggv_hbm, acc, gi, di, buf, sem):
        c = lax.axis_index("c")
        s = lax.axis_index("s")
        wid = c * NS + s
        pltpu.sync_copy(didx_hbm.at[wid], di)
        for p in range(4):
            pltpu.sync_copy(zrow_hbm, buf)
            for kk in range(CHUNKS):
                pltpu.sync_copy(buf, acc.at[pl.ds(s * RPT + kk * ZCH, ZCH)])
            pltpu.sync_copy(gidx_hbm.at[p, wid], gi)
            plsc.subcore_barrier()
            tbl = x_hbm if p == 0 else vf_hbm

            def body(b, carry):
                pltpu.async_copy(tbl.at[gi.at[b]], buf, sem).wait()
                pltpu.sync_copy(buf, acc.at[di.at[b]], add=True)
                return carry

            lax.fori_loop(0, NB, body, 0, unroll=4)
            plsc.subcore_barrier()
            for kk in range(CHUNKS):
                r = s * RPT + kk * ZCH
                pltpu.sync_copy(acc.at[pl.ds(r, ZCH)], buf)
                if p == 0:
                    pltpu.sync_copy(buf, aggx_hbm.at[c, pl.ds(r, ZCH)])
                else:
                    pltpu.sync_copy(buf, aggv_hbm.at[c, p - 1, pl.ds(r, ZCH)])
            plsc.subcore_barrier()

    return k(x, vecflat, gidx, didx, zrow, dpad)


def _tc_typed_matmul(an2, aggx_p, aggv_p, W_s, W_v):
    def body(an_ref, ax_ref, av_ref, ws_ref, wv_ref, s_out, v_out):
        f32 = jnp.float32
        an = an_ref[...]
        masks = [(an == t).astype(f32) for t in range(T)]
        ax = ax_ref[0] + ax_ref[1]
        s_acc = jnp.zeros((BN, D), f32)
        for t in range(T):
            s_acc += jnp.dot(ax * masks[t], ws_ref[t],
                             preferred_element_type=f32)
        s_out[...] = s_acc
        for comp in range(3):
            a = av_ref[0, comp] + av_ref[1, comp]
            acc = jnp.zeros((BN, D), f32)
            for t in range(T):
                acc += jnp.dot(a * masks[t], wv_ref[t],
                               preferred_element_type=f32)
            v_out[comp] = acc

    return pl.pallas_call(
        body,
        grid=(N // BN,),
        in_specs=[
            pl.BlockSpec((BN, 1), lambda i: (i, 0)),
            pl.BlockSpec((NC, BN, D), lambda i: (0, i, 0)),
            pl.BlockSpec((NC, 3, BN, D), lambda i: (0, 0, i, 0)),
            pl.BlockSpec((T, D, D), lambda i: (0, 0, 0)),
            pl.BlockSpec((T, D, D), lambda i: (0, 0, 0)),
        ],
        out_specs=[
            pl.BlockSpec((BN, D), lambda i: (i, 0)),
            pl.BlockSpec((3, BN, D), lambda i: (0, i, 0)),
        ],
        out_shape=[
            jax.ShapeDtypeStruct((N, D), jnp.float32),
            jax.ShapeDtypeStruct((3, N, D), jnp.float32),
        ],
    )(an2, aggx_p, aggv_p, W_s, W_v)


def kernel(x, vec, edge_index, atomic_number, W_s, W_v):
    src = edge_index[0]
    dst = edge_index[1]
    pad = EPAD - E
    srcp = jnp.concatenate([src, jnp.zeros((pad,), jnp.int32)])
    dstp = jnp.concatenate([dst, jnp.full((pad,), N, jnp.int32)])
    s3 = srcp * 3
    gidx = jnp.stack([srcp, s3, s3 + 1, s3 + 2]).reshape(4, NW, NB, BATCH)
    didx = dstp.reshape(NW, NB, BATCH)
    dpad = jnp.arange(N, N + BATCH, dtype=jnp.int32).reshape(1, BATCH)
    vecflat = vec.reshape(N * 3, D)
    zrow = jnp.zeros((ZCH, D), jnp.float32)

    aggx_p, aggv_p = _sc_aggregate(x, vecflat, gidx, didx, zrow, dpad)

    an2 = atomic_number.reshape(N, 1)
    srsts, v3 = _tc_typed_matmul(an2, aggx_p, aggv_p, W_s, W_v)
    vrsts = v3.transpose(1, 0, 2)
    return vrsts, srsts


# speedup vs baseline: 1.1453x; 1.1453x over previous
import functools

import jax
import jax.numpy as jnp
from jax import lax
from jax.experimental import pallas as pl
from jax.experimental.pallas import tpu as pltpu
from jax.experimental.pallas import tpu_sc as plsc

N = 10000
E = 320000
D = 128
T = 4

NC = 2
NS = 16
NW = NC * NS
BATCH = 128
NB = -(-E // (NW * BATCH))
EPW = NB * BATCH
EPAD = EPW * NW
ZCH = 128
RPT = -(-N // (NS * ZCH)) * ZCH
NPAD = RPT * NS
CHUNKS = RPT // ZCH

BN = 400


def _sc_aggregate(x, vecflat, gidx, didx, zrow, dpad):
    @functools.partial(
        pl.kernel,
        out_type=(
            jax.ShapeDtypeStruct((NC, NPAD, D), jnp.float32),
            jax.ShapeDtypeStruct((NC, 3, NPAD, D), jnp.float32),
        ),
        mesh=plsc.VectorSubcoreMesh(
            core_axis_name="c", subcore_axis_name="s",
            num_cores=NC, num_subcores=NS),
        scratch_types=[
            pltpu.VMEM_SHARED((NPAD, D), jnp.float32),
            pltpu.VMEM((NB, BATCH), jnp.int32),
            pltpu.VMEM((NB, BATCH), jnp.int32),
            pltpu.VMEM((BATCH, D), jnp.float32),
            pltpu.SemaphoreType.DMA,
            pltpu.SemaphoreType.DMA,
        ],
    )
    def k(x_hbm, vf_hbm, gidx_hbm, didx_hbm, zrow_hbm, dpad_hbm,
          aggx_hbm, aggv_hbm, acc, gi, di, buf, sem, ssem):
        c = lax.axis_index("c")
        s = lax.axis_index("s")
        wid = c * NS + s
        pltpu.sync_copy(didx_hbm.at[wid], di)
        for p in range(4):
            pltpu.sync_copy(zrow_hbm, buf)
            for kk in range(CHUNKS):
                pltpu.sync_copy(buf, acc.at[pl.ds(s * RPT + kk * ZCH, ZCH)])
            pltpu.sync_copy(gidx_hbm.at[p, wid], gi)
            plsc.subcore_barrier()
            tbl = x_hbm if p == 0 else vf_hbm

            def body(b, carry):
                pltpu.async_copy(tbl.at[gi.at[b]], buf, sem).wait()
                pltpu.async_copy(buf, acc.at[di.at[b]], ssem, add=True)
                return carry

            lax.fori_loop(0, NB, body, 0)

            def drain(b, carry):
                pltpu.make_async_copy(buf, acc.at[di.at[0]], ssem).wait()
                return carry

            lax.fori_loop(0, NB, drain, 0)
            plsc.subcore_barrier()
            for kk in range(CHUNKS):
                r = s * RPT + kk * ZCH
                pltpu.sync_copy(acc.at[pl.ds(r, ZCH)], buf)
                if p == 0:
                    pltpu.sync_copy(buf, aggx_hbm.at[c, pl.ds(r, ZCH)])
                else:
                    pltpu.sync_copy(buf, aggv_hbm.at[c, p - 1, pl.ds(r, ZCH)])
            plsc.subcore_barrier()

    return k(x, vecflat, gidx, didx, zrow, dpad)


def _tc_typed_matmul(an2, aggx_p, aggv_p, W_s, W_v):
    def body(an_ref, ax_ref, av_ref, ws_ref, wv_ref, s_out, v_out):
        f32 = jnp.float32
        an = an_ref[...]
        masks = [(an == t).astype(f32) for t in range(T)]
        ax = ax_ref[0] + ax_ref[1]
        s_acc = jnp.zeros((BN, D), f32)
        for t in range(T):
            s_acc += jnp.dot(ax * masks[t], ws_ref[t],
                             preferred_element_type=f32)
        s_out[...] = s_acc
        for comp in range(3):
            a = av_ref[0, comp] + av_ref[1, comp]
            acc = jnp.zeros((BN, D), f32)
            for t in range(T):
                acc += jnp.dot(a * masks[t], wv_ref[t],
                               preferred_element_type=f32)
            v_out[comp] = acc

    return pl.pallas_call(
        body,
        grid=(N // BN,),
        in_specs=[
            pl.BlockSpec((BN, 1), lambda i: (i, 0)),
            pl.BlockSpec((NC, BN, D), lambda i: (0, i, 0)),
            pl.BlockSpec((NC, 3, BN, D), lambda i: (0, 0, i, 0)),
            pl.BlockSpec((T, D, D), lambda i: (0, 0, 0)),
            pl.BlockSpec((T, D, D), lambda i: (0, 0, 0)),
        ],
        out_specs=[
            pl.BlockSpec((BN, D), lambda i: (i, 0)),
            pl.BlockSpec((3, BN, D), lambda i: (0, i, 0)),
        ],
        out_shape=[
            jax.ShapeDtypeStruct((N, D), jnp.float32),
            jax.ShapeDtypeStruct((3, N, D), jnp.float32),
        ],
    )(an2, aggx_p, aggv_p, W_s, W_v)


def kernel(x, vec, edge_index, atomic_number, W_s, W_v):
    src = edge_index[0]
    dst = edge_index[1]
    pad = EPAD - E
    srcp = jnp.concatenate([src, jnp.zeros((pad,), jnp.int32)])
    dstp = jnp.concatenate([dst, jnp.full((pad,), N, jnp.int32)])
    s3 = srcp * 3
    gidx = jnp.stack([srcp, s3, s3 + 1, s3 + 2]).reshape(4, NW, NB, BATCH)
    didx = dstp.reshape(NW, NB, BATCH)
    dpad = jnp.arange(N, N + BATCH, dtype=jnp.int32).reshape(1, BATCH)
    vecflat = vec.reshape(N * 3, D)
    zrow = jnp.zeros((ZCH, D), jnp.float32)

    aggx_p, aggv_p = _sc_aggregate(x, vecflat, gidx, didx, zrow, dpad)

    an2 = atomic_number.reshape(N, 1)
    srsts, v3 = _tc_typed_matmul(an2, aggx_p, aggv_p, W_s, W_v)
    vrsts = v3.transpose(1, 0, 2)
    return vrsts, srsts


# speedup vs baseline: 1.1460x; 1.0006x over previous
import functools

import jax
import jax.numpy as jnp
from jax import lax
from jax.experimental import pallas as pl
from jax.experimental.pallas import tpu as pltpu
from jax.experimental.pallas import tpu_sc as plsc

N = 10000
E = 320000
D = 128
T = 4

NC = 2
NS = 16
NW = NC * NS
BATCH = 128
NB = -(-E // (NW * BATCH))
EPW = NB * BATCH
EPAD = EPW * NW
ZCH = 128
RPT = -(-N // (NS * ZCH)) * ZCH
NPAD = RPT * NS
CHUNKS = RPT // ZCH

BN = 400


def _sc_aggregate(x, vecflat, gidx, didx, zrow):
    @functools.partial(
        pl.kernel,
        out_type=(
            jax.ShapeDtypeStruct((NC, NPAD, D), jnp.float32),
            jax.ShapeDtypeStruct((NC, 3, NPAD, D), jnp.float32),
        ),
        mesh=plsc.VectorSubcoreMesh(
            core_axis_name="c", subcore_axis_name="s",
            num_cores=NC, num_subcores=NS),
        scratch_types=[
            pltpu.VMEM_SHARED((NPAD, D), jnp.float32),
            pltpu.VMEM((NB, BATCH), jnp.int32),
            pltpu.VMEM((NB, BATCH), jnp.int32),
            pltpu.VMEM((BATCH, D), jnp.float32),
            pltpu.SemaphoreType.DMA,
            pltpu.SemaphoreType.DMA,
        ],
    )
    def k(x_hbm, vf_hbm, gidx_hbm, didx_hbm, zrow_hbm,
          aggx_hbm, aggv_hbm, acc, gi, di, buf, sem, ssem):
        c = lax.axis_index("c")
        s = lax.axis_index("s")
        wid = c * NS + s
        pltpu.sync_copy(didx_hbm.at[wid], di)
        for p in range(4):
            pltpu.sync_copy(zrow_hbm, buf)
            for kk in range(CHUNKS):
                pltpu.sync_copy(buf, acc.at[pl.ds(s * RPT + kk * ZCH, ZCH)])
            pltpu.sync_copy(gidx_hbm.at[p, wid], gi)
            plsc.subcore_barrier()
            tbl = x_hbm if p == 0 else vf_hbm

            def body(b, carry):
                pltpu.async_copy(tbl.at[gi.at[b]], buf, sem).wait()
                pltpu.async_copy(buf, acc.at[di.at[b]], ssem, add=True)
                return carry

            lax.fori_loop(0, NB, body, 0)

            def drain(b, carry):
                pltpu.make_async_copy(buf, acc.at[di.at[0]], ssem).wait()
                return carry

            lax.fori_loop(0, NB, drain, 0)
            plsc.subcore_barrier()
            for kk in range(CHUNKS):
                r = s * RPT + kk * ZCH
                pltpu.sync_copy(acc.at[pl.ds(r, ZCH)], buf)
                if p == 0:
                    pltpu.sync_copy(buf, aggx_hbm.at[c, pl.ds(r, ZCH)])
                else:
                    pltpu.sync_copy(buf, aggv_hbm.at[c, p - 1, pl.ds(r, ZCH)])
            plsc.subcore_barrier()

    return k(x, vecflat, gidx, didx, zrow)


def _tc_typed_matmul(an2, aggx_p, aggv_p, W_s, W_v):
    def body(an_ref, ax_ref, av_ref, ws_ref, wv_ref, s_out, v_out):
        f32 = jnp.float32
        an = an_ref[...]
        masks = [(an == t).astype(f32) for t in range(T)]
        ax = ax_ref[0] + ax_ref[1]
        s_acc = jnp.zeros((BN, D), f32)
        for t in range(T):
            s_acc += jnp.dot(ax * masks[t], ws_ref[t],
                             preferred_element_type=f32)
        s_out[...] = s_acc
        for comp in range(3):
            a = av_ref[0, comp] + av_ref[1, comp]
            acc = jnp.zeros((BN, D), f32)
            for t in range(T):
                acc += jnp.dot(a * masks[t], wv_ref[t],
                               preferred_element_type=f32)
            v_out[comp] = acc

    return pl.pallas_call(
        body,
        grid=(N // BN,),
        in_specs=[
            pl.BlockSpec((BN, 1), lambda i: (i, 0)),
            pl.BlockSpec((NC, BN, D), lambda i: (0, i, 0)),
            pl.BlockSpec((NC, 3, BN, D), lambda i: (0, 0, i, 0)),
            pl.BlockSpec((T, D, D), lambda i: (0, 0, 0)),
            pl.BlockSpec((T, D, D), lambda i: (0, 0, 0)),
        ],
        out_specs=[
            pl.BlockSpec((BN, D), lambda i: (i, 0)),
            pl.BlockSpec((3, BN, D), lambda i: (0, i, 0)),
        ],
        out_shape=[
            jax.ShapeDtypeStruct((N, D), jnp.float32),
            jax.ShapeDtypeStruct((3, N, D), jnp.float32),
        ],
    )(an2, aggx_p, aggv_p, W_s, W_v)


def kernel(x, vec, edge_index, atomic_number, W_s, W_v):
    src = edge_index[0]
    dst = edge_index[1]
    pad = EPAD - E
    srcp = jnp.concatenate([src, jnp.zeros((pad,), jnp.int32)])
    dstp = jnp.concatenate([dst, jnp.full((pad,), N, jnp.int32)])
    s3 = srcp * 3
    gidx = jnp.stack([srcp, s3, s3 + 1, s3 + 2]).reshape(4, NW, NB, BATCH)
    didx = dstp.reshape(NW, NB, BATCH)
    vecflat = vec.reshape(N * 3, D)
    zrow = jnp.zeros((ZCH, D), jnp.float32)

    aggx_p, aggv_p = _sc_aggregate(x, vecflat, gidx, didx, zrow)

    an2 = atomic_number.reshape(N, 1)
    srsts, v3 = _tc_typed_matmul(an2, aggx_p, aggv_p, W_s, W_v)
    vrsts = v3.transpose(1, 0, 2)
    return vrsts, srsts
